# baseline (speedup 1.0000x reference)
import jax
import jax.numpy as jnp
from jax import lax
from jax.experimental import pallas as pl
from jax.experimental.pallas import tpu as pltpu
from jax.experimental.pallas import tpu_sc as plsc

_L = 16
_NW = 32
_BIG = 2**30


def _sc_body(x_hbm, mask_hbm, zeros_hbm, out_hbm,
             mask_v, aidx, iidx, zbuf, dbuf, gsem, ssem):
    nrows = x_hbm.shape[0]
    srows = mask_hbm.shape[0]
    rpw = nrows // _NW
    wpb = srows // rpw
    wid = lax.axis_index("s") * 2 + lax.axis_index("c")
    base = wid * rpw
    mb = lax.rem(wid, wpb) * rpw

    pltpu.sync_copy(mask_hbm.at[pl.ds(mb, rpw)], mask_v)
    pltpu.sync_copy(zeros_hbm, zbuf)

    iota = lax.iota(jnp.int32, _L)

    def part_body(j, carry):
        na, ni, z0 = carry
        mv = mask_v[pl.ds(j * _L, _L)]
        act = mv != 0.0
        gidx = base + j * _L + iota
        acti = act.astype(jnp.int32)
        cs = jnp.cumsum(acti)
        plsc.store_scatter(aidx, [na + cs - 1], gidx, mask=act)
        plsc.store_scatter(iidx, [ni + iota - cs], gidx,
                           mask=jnp.logical_not(act))
        cnt = jnp.sum(acti)
        big = jnp.full((_L,), _BIG, jnp.int32)
        z0 = jnp.minimum(z0, jnp.min(jnp.where(act, big, gidx)))
        return na + cnt, ni + (_L - cnt), z0

    na, ni, z0 = lax.fori_loop(
        0, rpw // _L, part_body,
        (jnp.int32(0), jnp.int32(0), jnp.int32(_BIG)))

    zv = jnp.zeros((_L,), jnp.int32) + z0
    plsc.store_scatter(aidx, [na + iota], zv)
    plsc.store_scatter(iidx, [ni + iota], zv)

    nac = lax.shift_right_logical(na + (_L - 1), 4)
    nic = lax.shift_right_logical(ni + (_L - 1), 4)

    def act_body(c, carry):
        civ = aidx[pl.ds(c * _L, _L)]
        pltpu.async_copy(x_hbm.at[civ], dbuf, gsem).wait()
        pltpu.async_copy(dbuf, out_hbm.at[civ], ssem).wait()
        return carry

    lax.fori_loop(0, nac, act_body, jnp.int32(0))

    def zero_body(c, carry):
        civ = iidx[pl.ds(c * _L, _L)]
        pltpu.async_copy(zbuf, out_hbm.at[civ], ssem).wait()
        return carry

    lax.fori_loop(0, nic, zero_body, jnp.int32(0))


def kernel(x, mask):
    B, S, D = x.shape
    N = B * S
    x2 = x.reshape(N, D)
    zeros = jnp.zeros((_L, D), x.dtype)
    rpw = N // _NW
    mesh = plsc.VectorSubcoreMesh(core_axis_name="c", subcore_axis_name="s")
    k = pl.kernel(
        _sc_body,
        out_type=jax.ShapeDtypeStruct((N, D), x.dtype),
        mesh=mesh,
        compiler_params=pltpu.CompilerParams(needs_layout_passes=False),
        scratch_types=[
            pltpu.VMEM((rpw,), jnp.float32),
            pltpu.VMEM((rpw + _L,), jnp.int32),
            pltpu.VMEM((rpw + _L,), jnp.int32),
            pltpu.VMEM((_L, D), jnp.float32),
            pltpu.VMEM((_L, D), jnp.float32),
            pltpu.SemaphoreType.DMA,
            pltpu.SemaphoreType.DMA,
        ],
    )
    out = k(x2, mask, zeros)
    return out.reshape(B, S, D)


# speedup vs baseline: 1.0770x; 1.0770x over previous
import jax
import jax.numpy as jnp
from jax import lax
from jax.experimental import pallas as pl
from jax.experimental.pallas import tpu as pltpu
from jax.experimental.pallas import tpu_sc as plsc

_L = 16
_NW = 32
_BIG = 2**30


def _sc_body(x_hbm, mask_hbm, zeros_hbm, out_hbm,
             mask_v, aidx, iidx, zbuf, bufa, bufb,
             gsema, gsemb, ssema, ssemb, zsem):
    nrows = x_hbm.shape[0]
    srows = mask_hbm.shape[0]
    rpw = nrows // _NW
    wpb = srows // rpw
    wid = lax.axis_index("s") * 2 + lax.axis_index("c")
    base = wid * rpw
    mb = lax.rem(wid, wpb) * rpw

    pltpu.sync_copy(mask_hbm.at[pl.ds(mb, rpw)], mask_v)
    pltpu.sync_copy(zeros_hbm, zbuf)

    iota = lax.iota(jnp.int32, _L)

    def part_body(j, carry):
        na, ni, a0, z0 = carry
        mv = mask_v[pl.ds(j * _L, _L)]
        act = mv != 0.0
        gidx = base + j * _L + iota
        acti = act.astype(jnp.int32)
        cs = jnp.cumsum(acti)
        oa = na + cs - 1
        oi = ni + iota - cs
        plsc.store_scatter(aidx, [lax.shift_right_arithmetic(oa, 4), oa & (_L - 1)],
                           gidx, mask=act)
        plsc.store_scatter(iidx, [lax.shift_right_arithmetic(oi, 4), oi & (_L - 1)],
                           gidx, mask=jnp.logical_not(act))
        cnt = jnp.sum(acti)
        big = jnp.full((_L,), _BIG, jnp.int32)
        a0 = jnp.minimum(a0, jnp.min(jnp.where(act, gidx, big)))
        z0 = jnp.minimum(z0, jnp.min(jnp.where(act, big, gidx)))
        return na + cnt, ni + (_L - cnt), a0, z0

    na, ni, a0, z0 = lax.fori_loop(
        0, rpw // _L, part_body,
        (jnp.int32(0), jnp.int32(0), jnp.int32(_BIG), jnp.int32(_BIG)))

    zeros16 = jnp.zeros((_L,), jnp.int32)
    for off in (0, _L):
        oa = na + off + iota
        oi = ni + off + iota
        plsc.store_scatter(aidx, [lax.shift_right_arithmetic(oa, 4), oa & (_L - 1)],
                           zeros16 + a0)
        plsc.store_scatter(iidx, [lax.shift_right_arithmetic(oi, 4), oi & (_L - 1)],
                           zeros16 + z0)

    nic = lax.shift_right_logical(ni + (_L - 1), 4)
    nap = lax.shift_right_logical(na + (2 * _L - 1), 5)

    def zfire(c, carry):
        pltpu.async_copy(zbuf, out_hbm.at[iidx.at[c]], zsem)
        return carry

    lax.fori_loop(0, nic, zfire, jnp.int32(0))

    def act_body(p, carry):
        @pl.when(p > 0)
        def _():
            pltpu.make_async_copy(bufa, out_hbm.at[aidx.at[0]], ssema).wait()
            pltpu.make_async_copy(bufb, out_hbm.at[aidx.at[0]], ssemb).wait()
        ca = pltpu.async_copy(x_hbm.at[aidx.at[2 * p]], bufa, gsema)
        cb = pltpu.async_copy(x_hbm.at[aidx.at[2 * p + 1]], bufb, gsemb)
        ca.wait()
        pltpu.async_copy(bufa, out_hbm.at[aidx.at[2 * p]], ssema)
        cb.wait()
        pltpu.async_copy(bufb, out_hbm.at[aidx.at[2 * p + 1]], ssemb)
        return carry

    lax.fori_loop(0, nap, act_body, jnp.int32(0))

    @pl.when(nap > 0)
    def _():
        pltpu.make_async_copy(bufa, out_hbm.at[aidx.at[0]], ssema).wait()
        pltpu.make_async_copy(bufb, out_hbm.at[aidx.at[0]], ssemb).wait()

    def zdrain(c, carry):
        pltpu.make_async_copy(zbuf, out_hbm.at[iidx.at[0]], zsem).wait()
        return carry

    lax.fori_loop(0, nic, zdrain, jnp.int32(0))


def kernel(x, mask):
    B, S, D = x.shape
    N = B * S
    x2 = x.reshape(N, D)
    zeros = jnp.zeros((_L, D), x.dtype)
    rpw = N // _NW
    nch = rpw // _L + 2
    mesh = plsc.VectorSubcoreMesh(core_axis_name="c", subcore_axis_name="s")
    k = pl.kernel(
        _sc_body,
        out_type=jax.ShapeDtypeStruct((N, D), x.dtype),
        mesh=mesh,
        compiler_params=pltpu.CompilerParams(needs_layout_passes=False),
        scratch_types=[
            pltpu.VMEM((rpw,), jnp.float32),
            pltpu.VMEM((nch, _L), jnp.int32),
            pltpu.VMEM((nch, _L), jnp.int32),
            pltpu.VMEM((_L, D), jnp.float32),
            pltpu.VMEM((_L, D), jnp.float32),
            pltpu.VMEM((_L, D), jnp.float32),
            pltpu.SemaphoreType.DMA,
            pltpu.SemaphoreType.DMA,
            pltpu.SemaphoreType.DMA,
            pltpu.SemaphoreType.DMA,
            pltpu.SemaphoreType.DMA,
        ],
    )
    out = k(x2, mask, zeros)
    return out.reshape(B, S, D)


# speedup vs baseline: 1.0955x; 1.0172x over previous
import jax
import jax.numpy as jnp
from jax import lax
from jax.experimental import pallas as pl
from jax.experimental.pallas import tpu as pltpu
from jax.experimental.pallas import tpu_sc as plsc

_L = 16
_NW = 32
_CW = 8
_NB = 4
_BIG = 2**30


def _sc_body(x_hbm, mask_hbm, zeros_hbm, out_hbm,
             mask_v, aidx, iidx, zbuf, bufs, gsem, ssem, zsem):
    nrows = x_hbm.shape[0]
    srows = mask_hbm.shape[0]
    rpw = nrows // _NW
    wpb = srows // rpw
    wid = lax.axis_index("s") * 2 + lax.axis_index("c")
    base = wid * rpw
    mb = lax.rem(wid, wpb) * rpw

    pltpu.sync_copy(mask_hbm.at[pl.ds(mb, rpw)], mask_v)
    pltpu.sync_copy(zeros_hbm, zbuf)

    iota = lax.iota(jnp.int32, _L)

    def part_body(j, carry):
        na, ni, a0, z0 = carry
        mv = mask_v[pl.ds(j * _L, _L)]
        act = mv != 0.0
        gidx = base + j * _L + iota
        acti = act.astype(jnp.int32)
        cs = jnp.cumsum(acti)
        oa = na + cs - 1
        oi = ni + iota - cs
        plsc.store_scatter(
            aidx, [lax.shift_right_arithmetic(oa, 3), oa & (_CW - 1)],
            gidx, mask=act)
        plsc.store_scatter(
            iidx, [lax.shift_right_arithmetic(oi, 3), oi & (_CW - 1)],
            gidx, mask=jnp.logical_not(act))
        cnt = jnp.sum(acti)
        big = jnp.full((_L,), _BIG, jnp.int32)
        a0 = jnp.minimum(a0, jnp.min(jnp.where(act, gidx, big)))
        z0 = jnp.minimum(z0, jnp.min(jnp.where(act, big, gidx)))
        return na + cnt, ni + (_L - cnt), a0, z0

    na, ni, a0, z0 = lax.fori_loop(
        0, rpw // _L, part_body,
        (jnp.int32(0), jnp.int32(0), jnp.int32(_BIG), jnp.int32(_BIG)))

    zeros16 = jnp.zeros((_L,), jnp.int32)
    for off in (0, _L):
        oa = na + off + iota
        plsc.store_scatter(
            aidx, [lax.shift_right_arithmetic(oa, 3), oa & (_CW - 1)],
            zeros16 + a0)
    oi = ni + iota
    plsc.store_scatter(
        iidx, [lax.shift_right_arithmetic(oi, 3), oi & (_CW - 1)],
        zeros16 + z0)

    nic = lax.shift_right_logical(ni + (_CW - 1), 3)
    nat = lax.shift_right_logical(na + (_NB * _CW - 1), 5)

    def zfire(c, carry):
        pltpu.async_copy(zbuf, out_hbm.at[iidx.at[c]], zsem)
        return carry

    lax.fori_loop(0, nic, zfire, jnp.int32(0))

    def act_body(p, carry):
        c = _NB * p

        @pl.when(p > 0)
        def _():
            for q in range(_NB):
                pltpu.make_async_copy(
                    bufs.at[q], out_hbm.at[aidx.at[0]], ssem[q]).wait()
        handles = []
        for q in range(_NB):
            handles.append(pltpu.async_copy(
                x_hbm.at[aidx.at[c + q]], bufs.at[q], gsem[q]))
        for q in range(_NB):
            handles[q].wait()
            pltpu.async_copy(bufs.at[q], out_hbm.at[aidx.at[c + q]], ssem[q])
        return carry

    lax.fori_loop(0, nat, act_body, jnp.int32(0))

    @pl.when(nat > 0)
    def _():
        for q in range(_NB):
            pltpu.make_async_copy(
                bufs.at[q], out_hbm.at[aidx.at[0]], ssem[q]).wait()

    def zdrain(c, carry):
        pltpu.make_async_copy(zbuf, out_hbm.at[iidx.at[0]], zsem).wait()
        return carry

    lax.fori_loop(0, nic, zdrain, jnp.int32(0))


def kernel(x, mask):
    B, S, D = x.shape
    N = B * S
    x2 = x.reshape(N, D)
    zeros = jnp.zeros((_CW, D), x.dtype)
    rpw = N // _NW
    nch = rpw // _CW + 2
    mesh = plsc.VectorSubcoreMesh(core_axis_name="c", subcore_axis_name="s")
    k = pl.kernel(
        _sc_body,
        out_type=jax.ShapeDtypeStruct((N, D), x.dtype),
        mesh=mesh,
        compiler_params=pltpu.CompilerParams(needs_layout_passes=False),
        scratch_types=[
            pltpu.VMEM((rpw,), jnp.float32),
            pltpu.VMEM((nch, _CW), jnp.int32),
            pltpu.VMEM((nch, _CW), jnp.int32),
            pltpu.VMEM((_CW, D), jnp.float32),
            pltpu.VMEM((_NB, _CW, D), jnp.float32),
            [pltpu.SemaphoreType.DMA] * _NB,
            [pltpu.SemaphoreType.DMA] * _NB,
            pltpu.SemaphoreType.DMA,
        ],
    )
    out = k(x2, mask, zeros)
    return out.reshape(B, S, D)
